# baseline (speedup 1.0000x reference)
import functools

import numpy as np
import jax
import jax.numpy as jnp
from jax import lax
from jax.experimental import pallas as pl
from jax.experimental.pallas import tpu as pltpu
from jax.experimental.pallas import tpu_sc as plsc

A = 128
B = 1024
HID = 1024
EMB = 512
K = 8192
UC = 768
NAA = 2560
IN_AUG = UC + NAA

BB = 256
KB = 1024

f32 = jnp.float32
bf16 = jnp.bfloat16

_EE = np.zeros((UC, 384), np.float32)
_EO = np.zeros((UC, 384), np.float32)
_EE[np.arange(384) * 2, np.arange(384)] = 1.0
_EO[np.arange(384) * 2 + 1, np.arange(384)] = 1.0
_G20 = np.zeros((NAA, A), np.float32)
_G20[np.arange(NAA), np.arange(NAA) // 20] = 1.0


def _dgT(x, y):
    return lax.dot_general(x, y, (((1,), (1,)), ((), ())),
                           preferred_element_type=f32)


def _ci_body(seq_ref, ci_ref):
    s3 = seq_ref[...]
    m = jnp.max(s3, axis=1, keepdims=True)
    ii = lax.broadcasted_iota(jnp.int32, s3.shape, 1)
    ci = jnp.min(jnp.where(s3 == m, ii, 31), axis=1, keepdims=True)
    ci_ref[...] = ci.astype(f32)


def _encvq_body(ang_ref, w0_ref, eet_ref, eot_ref, b0_ref, w1_ref, b1_ref,
                w2_ref, b2_ref, cb_ref, c_ref, s_ref, e_ref, bi_ref,
                num_ref, wm_ref, cbbf_s, cn_s, cs_s, ss_s, es_s):
    i = pl.program_id(0)

    @pl.when(i == 0)
    def _():
        def prep(k, _):
            ck = cb_ref[pl.ds(k * 1024, 1024), :]
            cn_s[pl.ds(k * 1024, 1024), :] = jnp.sum(ck * ck, axis=1,
                                                     keepdims=True)
            cbbf_s[pl.ds(k * 1024, 1024), :] = ck.astype(bf16)
            return 0
        lax.fori_loop(0, K // 1024, prep, 0)

    a = ang_ref[...]
    c = jnp.cos(a)
    s = jnp.sin(a)
    c_ref[...] = c
    s_ref[...] = s
    cb_ = c.astype(bf16)
    sb_ = s.astype(bf16)
    csil = (jnp.dot(cb_, eet_ref[...], preferred_element_type=f32)
            + jnp.dot(sb_, eot_ref[...], preferred_element_type=f32))
    h = jnp.dot(csil.astype(bf16), w0_ref[...].astype(bf16),
                preferred_element_type=f32)
    h = jnp.maximum(h + b0_ref[...], 0.0)
    h = jnp.maximum(
        jnp.dot(h.astype(bf16), w1_ref[...].astype(bf16),
                preferred_element_type=f32) + b1_ref[...], 0.0)
    e = jnp.maximum(
        jnp.dot(h.astype(bf16), w2_ref[...].astype(bf16),
                preferred_element_type=f32) + b2_ref[...], 0.0)
    e_ref[...] = e
    eb = e.astype(bf16)

    ii = lax.broadcasted_iota(jnp.int32, (KB, e.shape[0]), 0)

    def chunk(k, carry):
        lm, la = carry
        ck = cbbf_s[pl.ds(k * KB, KB), :]
        cn = cn_s[pl.ds(k * KB, KB), :]
        sgn = cn - 2.0 * _dgT(ck, eb)
        cm = jnp.min(sgn, axis=0)
        cidx = jnp.min(jnp.where(sgn == cm[None, :], ii + k * KB, K), axis=0)
        upd = cm < lm
        return jnp.where(upd, cm, lm), jnp.where(upd, cidx, la)

    init = (jnp.full((e.shape[0],), jnp.inf, f32),
            jnp.zeros((e.shape[0],), jnp.int32))
    _, la = lax.fori_loop(0, K // KB, chunk, init)
    bi_ref[...] = la[None, :]

    cs_s[pl.ds(i * BB, BB), :] = cb_
    ss_s[pl.ds(i * BB, BB), :] = sb_
    es_s[pl.ds(i * BB, BB), :] = eb
    ebf = eb.astype(f32)
    e2i = jnp.sum(ebf * ebf, axis=1)

    def pair(j, carry):
        an, aw = carry
        Cj = cs_s[pl.ds(j * BB, BB), :]
        Sj = ss_s[pl.ds(j * BB, BB), :]
        Ej = es_s[pl.ds(j * BB, BB), :]
        fac = jnp.where(j == i, 1.0, 2.0)
        mc = (_dgT(cb_, Cj) + _dgT(sb_, Sj)) * (1.0 / 384.0)
        Dd = 0.5 * (1.0 - mc)
        Y = jnp.where(Dd < 0.1, 1.0, jnp.where(Dd > 0.47, 0.0, 0.5))
        wm = jnp.where(Y == 0.5, 0.0, 1.0)
        Ejf = Ej.astype(f32)
        e2j = jnp.sum(Ejf * Ejf, axis=1)
        d2 = jnp.maximum(
            e2i[:, None] + e2j[None, :] - 2.0 * _dgT(eb, Ej), 0.0)
        dn = jnp.sqrt(d2 + 1e-8)
        rl = jnp.maximum(1.0 - dn, 0.0)
        t = wm * (Y * (d2 + 1e-8) + (1.0 - Y) * rl * rl)
        return an + fac * jnp.sum(t), aw + fac * jnp.sum(wm)

    an, aw = lax.fori_loop(0, i + 1, pair,
                           (jnp.float32(0.0), jnp.float32(0.0)))

    @pl.when(i == 0)
    def _():
        num_ref[...] = an.reshape(1, 1)
        wm_ref[...] = aw.reshape(1, 1)

    @pl.when(i != 0)
    def _():
        num_ref[...] = num_ref[...] + an.reshape(1, 1)
        wm_ref[...] = wm_ref[...] + aw.reshape(1, 1)


def _dec_body(q_ref, e_ref, c_ref, s_ref, ci_ref,
              w0_ref, b0_ref, w1_ref, b1_ref, wuc_ref, buc_ref,
              waa_ref, baa_ref, ee_ref, eo_ref, g20_ref, g20t_ref,
              rec_ref, ucr_ref, com_ref, aa_ref):
    first = pl.program_id(0) == 0
    q = q_ref[...]
    h = jnp.maximum(
        jnp.dot(q.astype(bf16), w0_ref[...].astype(bf16),
                preferred_element_type=f32) + b0_ref[...], 0.0)
    h = jnp.maximum(
        jnp.dot(h.astype(bf16), w1_ref[...].astype(bf16),
                preferred_element_type=f32) + b1_ref[...], 0.0)
    hb = h.astype(bf16)
    duc = (jnp.dot(hb, wuc_ref[...].astype(bf16), preferred_element_type=f32)
           + buc_ref[...])
    aa = (jnp.dot(hb, waa_ref[...].astype(bf16), preferred_element_type=f32)
          + baa_ref[...])
    ducb = duc.astype(bf16)
    xr = jnp.dot(ducb, ee_ref[...], preferred_element_type=f32)
    yr = jnp.dot(ducb, eo_ref[...], preferred_element_type=f32)
    C = c_ref[...]
    S = s_ref[...]
    rec = (jnp.sum((xr - C) ** 2) + jnp.sum((yr - S) ** 2)
           + jnp.sum(aa * aa))
    r2 = xr * xr + yr * yr
    ucr = jnp.sum((r2 - 1.0) ** 2)
    com = jnp.sum((e_ref[...] - q) ** 2)
    rm = jnp.max(aa, axis=1, keepdims=True)
    ex = jnp.exp(aa - rm)
    se = jnp.dot(ex.astype(bf16), g20_ref[...], preferred_element_type=f32)
    lse_sum = jnp.sum(jnp.log(se)) + jnp.sum(rm) * A
    ci = ci_ref[...]
    ciexp = jnp.dot(ci.astype(bf16), g20t_ref[...],
                    preferred_element_type=f32)
    colpat = (lax.broadcasted_iota(jnp.int32, aa.shape, 1) % 20).astype(f32)
    sel_sum = jnp.sum(jnp.where(colpat == ciexp, aa, 0.0))
    aas = sel_sum - lse_sum

    rec = rec.reshape(1, 1)
    ucr = ucr.reshape(1, 1)
    com = com.reshape(1, 1)
    aas = aas.reshape(1, 1)

    @pl.when(first)
    def _():
        rec_ref[...] = rec
        ucr_ref[...] = ucr
        com_ref[...] = com
        aa_ref[...] = aas

    @pl.when(~first)
    def _():
        rec_ref[...] = rec_ref[...] + rec
        ucr_ref[...] = ucr_ref[...] + ucr
        com_ref[...] = com_ref[...] + com
        aa_ref[...] = aa_ref[...] + aas


def _whole(shape):
    return pl.BlockSpec(shape, lambda *_: tuple(0 for _ in shape))


def _sc_gather(table, idx):
    info = plsc.get_sparse_core_info()
    nw = info.num_cores * info.num_subcores
    bpw = B // nw
    mesh = plsc.VectorSubcoreMesh(core_axis_name="c", subcore_axis_name="s")

    @functools.partial(
        pl.kernel, mesh=mesh,
        out_type=jax.ShapeDtypeStruct((B, EMB), f32),
        scratch_types=[
            pltpu.VMEM((bpw,), jnp.int32),
            pltpu.VMEM((bpw, EMB), f32),
            pltpu.SemaphoreType.DMA,
        ],
    )
    def k(table_hbm, idx_hbm, out_hbm, idx_v, rows_v, sem):
        wid = lax.axis_index("s") * info.num_cores + lax.axis_index("c")
        base = wid * bpw
        pltpu.sync_copy(idx_hbm.at[pl.ds(base, bpw)], idx_v)
        pltpu.async_copy(table_hbm.at[idx_v], rows_v, sem).wait()
        pltpu.sync_copy(rows_v, out_hbm.at[pl.ds(base, bpw)])

    return k(table, idx)


def kernel(angles, sequence, W_e0, b_e0, W_e1, b_e1, W_e2, b_e2,
           W_d0, b_d0, W_d1, b_d1, W_d2, b_d2, codebook):
    nb = B // BB
    wuc = W_d2[:, :UC]
    waa = W_d2[:, UC:]
    buc = b_d2[:UC].reshape(1, UC)
    baa = b_d2[UC:].reshape(1, NAA)
    seq3 = sequence.reshape(B * A, 20)
    b0 = b_e0.reshape(1, HID)
    b1 = b_e1.reshape(1, HID)
    b2 = b_e2.reshape(1, EMB)
    bd0 = b_d0.reshape(1, HID)
    bd1 = b_d1.reshape(1, HID)
    ee = jnp.asarray(_EE, bf16)
    eo = jnp.asarray(_EO, bf16)
    eet = jnp.asarray(_EE.T, bf16)
    eot = jnp.asarray(_EO.T, bf16)
    g20 = jnp.asarray(_G20, bf16)
    g20t = jnp.asarray(_G20.T, bf16)

    CIB = 8192
    ci3 = pl.pallas_call(
        _ci_body,
        grid=(B * A // CIB,),
        in_specs=[pl.BlockSpec((CIB, 20), lambda i: (i, 0))],
        out_specs=pl.BlockSpec((CIB, 1), lambda i: (i, 0)),
        out_shape=jax.ShapeDtypeStruct((B * A, 1), f32),
    )(seq3)
    ci = ci3.reshape(B, A)

    C, S, enc, bi, num, wsum = pl.pallas_call(
        _encvq_body,
        grid=(nb,),
        in_specs=[
            pl.BlockSpec((BB, 384), lambda i: (i, 0)),
            pl.BlockSpec((UC, HID), lambda i: (0, 0)),
            _whole((384, UC)), _whole((384, UC)), _whole((1, HID)),
            _whole((HID, HID)), _whole((1, HID)),
            _whole((HID, EMB)), _whole((1, EMB)),
            _whole((K, EMB)),
        ],
        out_specs=[
            pl.BlockSpec((BB, 384), lambda i: (i, 0)),
            pl.BlockSpec((BB, 384), lambda i: (i, 0)),
            pl.BlockSpec((BB, EMB), lambda i: (i, 0)),
            pl.BlockSpec((1, BB), lambda i: (0, i)),
            pl.BlockSpec((1, 1), lambda i: (0, 0)),
            pl.BlockSpec((1, 1), lambda i: (0, 0)),
        ],
        out_shape=[
            jax.ShapeDtypeStruct((B, 384), f32),
            jax.ShapeDtypeStruct((B, 384), f32),
            jax.ShapeDtypeStruct((B, EMB), f32),
            jax.ShapeDtypeStruct((1, B), jnp.int32),
            jax.ShapeDtypeStruct((1, 1), f32),
            jax.ShapeDtypeStruct((1, 1), f32),
        ],
        scratch_shapes=[
            pltpu.VMEM((K, EMB), bf16),
            pltpu.VMEM((K, 1), f32),
            pltpu.VMEM((B, 384), bf16),
            pltpu.VMEM((B, 384), bf16),
            pltpu.VMEM((B, EMB), bf16),
        ],
    )(angles, W_e0, eet, eot, b0, W_e1, b1, W_e2, b2, codebook)

    quant = _sc_gather(codebook, bi[0, :])

    rec, ucr, com, aas = pl.pallas_call(
        _dec_body,
        grid=(nb,),
        in_specs=[
            pl.BlockSpec((BB, EMB), lambda i: (i, 0)),
            pl.BlockSpec((BB, EMB), lambda i: (i, 0)),
            pl.BlockSpec((BB, 384), lambda i: (i, 0)),
            pl.BlockSpec((BB, 384), lambda i: (i, 0)),
            pl.BlockSpec((BB, A), lambda i: (i, 0)),
            _whole((EMB, HID)), _whole((1, HID)),
            _whole((HID, HID)), _whole((1, HID)),
            _whole((HID, UC)), _whole((1, UC)),
            _whole((HID, NAA)), _whole((1, NAA)),
            _whole((UC, 384)), _whole((UC, 384)),
            _whole((NAA, A)), _whole((A, NAA)),
        ],
        out_specs=[pl.BlockSpec((1, 1), lambda i: (0, 0))] * 4,
        out_shape=[jax.ShapeDtypeStruct((1, 1), f32)] * 4,
    )(quant, enc, C, S, ci, W_d0, bd0, W_d1, bd1, wuc, buc,
      waa, baa, ee, eo, g20, g20t)

    recon = rec[0, 0] / (B * IN_AUG)
    commit = 0.25 * com[0, 0] / (B * EMB)
    aa_loss = -aas[0, 0] / (B * A)
    uc_reg = ucr[0, 0] / (B * 384)
    dih = num[0, 0] / jnp.maximum(wsum[0, 0], 1.0)
    return recon + commit + aa_loss + 0.01 * uc_reg + 0.1 * dih


# speedup vs baseline: 1.6401x; 1.6401x over previous
import functools

import numpy as np
import jax
import jax.numpy as jnp
from jax import lax
from jax.experimental import pallas as pl
from jax.experimental.pallas import tpu as pltpu
from jax.experimental.pallas import tpu_sc as plsc

A = 128
B = 1024
HID = 1024
EMB = 512
K = 8192
UC = 768
NAA = 2560
IN_AUG = UC + NAA

BB = 256
KB = 1024

f32 = jnp.float32
bf16 = jnp.bfloat16

_EE = np.zeros((UC, 384), np.float32)
_EO = np.zeros((UC, 384), np.float32)
_EE[np.arange(384) * 2, np.arange(384)] = 1.0
_EO[np.arange(384) * 2 + 1, np.arange(384)] = 1.0
_G20 = np.zeros((NAA, A), np.float32)
_G20[np.arange(NAA), np.arange(NAA) // 20] = 1.0


def _dgT(x, y):
    return lax.dot_general(x, y, (((1,), (1,)), ((), ())),
                           preferred_element_type=f32)


def _encvq_body(ang_ref, w0_ref, eet_ref, eot_ref, b0_ref, w1_ref, b1_ref,
                w2_ref, b2_ref, cb_ref, c_ref, s_ref, e_ref, bi_ref,
                num_ref, wm_ref, cbbf_s, cn_s, cs_s, ss_s, es_s):
    i = pl.program_id(0)

    @pl.when(i == 0)
    def _():
        def prep(k, _):
            ck = cb_ref[pl.ds(k * 1024, 1024), :]
            cn_s[pl.ds(k * 1024, 1024), :] = jnp.sum(ck * ck, axis=1,
                                                     keepdims=True)
            cbbf_s[pl.ds(k * 1024, 1024), :] = ck.astype(bf16)
            return 0
        lax.fori_loop(0, K // 1024, prep, 0)

    a = ang_ref[...]
    c = jnp.cos(a)
    s = jnp.sin(a)
    c_ref[...] = c
    s_ref[...] = s
    cb_ = c.astype(bf16)
    sb_ = s.astype(bf16)
    csil = (jnp.dot(cb_, eet_ref[...], preferred_element_type=f32)
            + jnp.dot(sb_, eot_ref[...], preferred_element_type=f32))
    h = jnp.dot(csil.astype(bf16), w0_ref[...].astype(bf16),
                preferred_element_type=f32)
    h = jnp.maximum(h + b0_ref[...], 0.0)
    h = jnp.maximum(
        jnp.dot(h.astype(bf16), w1_ref[...].astype(bf16),
                preferred_element_type=f32) + b1_ref[...], 0.0)
    e = jnp.maximum(
        jnp.dot(h.astype(bf16), w2_ref[...].astype(bf16),
                preferred_element_type=f32) + b2_ref[...], 0.0)
    e_ref[...] = e
    eb = e.astype(bf16)

    ii = lax.broadcasted_iota(jnp.int32, (KB, e.shape[0]), 0)

    def chunk(k, carry):
        lm, la = carry
        ck = cbbf_s[pl.ds(k * KB, KB), :]
        cn = cn_s[pl.ds(k * KB, KB), :]
        sgn = cn - 2.0 * _dgT(ck, eb)
        cm = jnp.min(sgn, axis=0)
        cidx = jnp.min(jnp.where(sgn == cm[None, :], ii + k * KB, K), axis=0)
        upd = cm < lm
        return jnp.where(upd, cm, lm), jnp.where(upd, cidx, la)

    init = (jnp.full((e.shape[0],), jnp.inf, f32),
            jnp.zeros((e.shape[0],), jnp.int32))
    _, la = lax.fori_loop(0, K // KB, chunk, init)
    bi_ref[...] = la[None, :]

    cs_s[pl.ds(i * BB, BB), :] = cb_
    ss_s[pl.ds(i * BB, BB), :] = sb_
    es_s[pl.ds(i * BB, BB), :] = eb
    ebf = eb.astype(f32)
    e2i = jnp.sum(ebf * ebf, axis=1)

    def pair(j, carry):
        an, aw = carry
        Cj = cs_s[pl.ds(j * BB, BB), :]
        Sj = ss_s[pl.ds(j * BB, BB), :]
        Ej = es_s[pl.ds(j * BB, BB), :]
        fac = jnp.where(j == i, 1.0, 2.0)
        mc = (_dgT(cb_, Cj) + _dgT(sb_, Sj)) * (1.0 / 384.0)
        Dd = 0.5 * (1.0 - mc)
        Y = jnp.where(Dd < 0.1, 1.0, jnp.where(Dd > 0.47, 0.0, 0.5))
        wm = jnp.where(Y == 0.5, 0.0, 1.0)
        Ejf = Ej.astype(f32)
        e2j = jnp.sum(Ejf * Ejf, axis=1)
        d2 = jnp.maximum(
            e2i[:, None] + e2j[None, :] - 2.0 * _dgT(eb, Ej), 0.0)
        dn = jnp.sqrt(d2 + 1e-8)
        rl = jnp.maximum(1.0 - dn, 0.0)
        t = wm * (Y * (d2 + 1e-8) + (1.0 - Y) * rl * rl)
        return an + fac * jnp.sum(t), aw + fac * jnp.sum(wm)

    an, aw = lax.fori_loop(0, i + 1, pair,
                           (jnp.float32(0.0), jnp.float32(0.0)))

    @pl.when(i == 0)
    def _():
        num_ref[...] = an.reshape(1, 1)
        wm_ref[...] = aw.reshape(1, 1)

    @pl.when(i != 0)
    def _():
        num_ref[...] = num_ref[...] + an.reshape(1, 1)
        wm_ref[...] = wm_ref[...] + aw.reshape(1, 1)


def _dec_body(q_ref, e_ref, c_ref, s_ref, ci_ref,
              w0_ref, b0_ref, w1_ref, b1_ref, wuc_ref, buc_ref,
              waa_ref, baa_ref, ee_ref, eo_ref, g20_ref, g20t_ref,
              rec_ref, ucr_ref, com_ref, aa_ref):
    first = pl.program_id(0) == 0
    q = q_ref[...]
    h = jnp.maximum(
        jnp.dot(q.astype(bf16), w0_ref[...].astype(bf16),
                preferred_element_type=f32) + b0_ref[...], 0.0)
    h = jnp.maximum(
        jnp.dot(h.astype(bf16), w1_ref[...].astype(bf16),
                preferred_element_type=f32) + b1_ref[...], 0.0)
    hb = h.astype(bf16)
    duc = (jnp.dot(hb, wuc_ref[...].astype(bf16), preferred_element_type=f32)
           + buc_ref[...])
    aa = (jnp.dot(hb, waa_ref[...].astype(bf16), preferred_element_type=f32)
          + baa_ref[...])
    ducb = duc.astype(bf16)
    xr = jnp.dot(ducb, ee_ref[...], preferred_element_type=f32)
    yr = jnp.dot(ducb, eo_ref[...], preferred_element_type=f32)
    C = c_ref[...]
    S = s_ref[...]
    rec = (jnp.sum((xr - C) ** 2) + jnp.sum((yr - S) ** 2)
           + jnp.sum(aa * aa))
    r2 = xr * xr + yr * yr
    ucr = jnp.sum((r2 - 1.0) ** 2)
    com = jnp.sum((e_ref[...] - q) ** 2)
    rm = jnp.max(aa, axis=1, keepdims=True)
    ex = jnp.exp(aa - rm)
    se = jnp.dot(ex.astype(bf16), g20_ref[...], preferred_element_type=f32)
    lse_sum = jnp.sum(jnp.log(se)) + jnp.sum(rm) * A
    seq = ci_ref[...]
    best = seq[:, 0:A]
    ci = jnp.zeros((seq.shape[0], A), bf16)
    for c in range(1, 20):
        sc = seq[:, c * A:(c + 1) * A]
        upd = sc > best
        best = jnp.where(upd, sc, best)
        ci = jnp.where(upd, jnp.bfloat16(c), ci)
    ciexp = jnp.dot(ci, g20t_ref[...],
                    preferred_element_type=f32)
    colpat = (lax.broadcasted_iota(jnp.int32, aa.shape, 1) % 20).astype(f32)
    sel_sum = jnp.sum(jnp.where(colpat == ciexp, aa, 0.0))
    aas = sel_sum - lse_sum

    rec = rec.reshape(1, 1)
    ucr = ucr.reshape(1, 1)
    com = com.reshape(1, 1)
    aas = aas.reshape(1, 1)

    @pl.when(first)
    def _():
        rec_ref[...] = rec
        ucr_ref[...] = ucr
        com_ref[...] = com
        aa_ref[...] = aas

    @pl.when(~first)
    def _():
        rec_ref[...] = rec_ref[...] + rec
        ucr_ref[...] = ucr_ref[...] + ucr
        com_ref[...] = com_ref[...] + com
        aa_ref[...] = aa_ref[...] + aas


def _whole(shape):
    return pl.BlockSpec(shape, lambda *_: tuple(0 for _ in shape))


def _sc_gather(table, idx):
    info = plsc.get_sparse_core_info()
    nw = info.num_cores * info.num_subcores
    bpw = B // nw
    mesh = plsc.VectorSubcoreMesh(core_axis_name="c", subcore_axis_name="s")

    @functools.partial(
        pl.kernel, mesh=mesh,
        out_type=jax.ShapeDtypeStruct((B, EMB), f32),
        scratch_types=[
            pltpu.VMEM((bpw,), jnp.int32),
            pltpu.VMEM((bpw, EMB), f32),
            pltpu.SemaphoreType.DMA,
        ],
    )
    def k(table_hbm, idx_hbm, out_hbm, idx_v, rows_v, sem):
        wid = lax.axis_index("s") * info.num_cores + lax.axis_index("c")
        base = wid * bpw
        pltpu.sync_copy(idx_hbm.at[pl.ds(base, bpw)], idx_v)
        pltpu.async_copy(table_hbm.at[idx_v], rows_v, sem).wait()
        pltpu.sync_copy(rows_v, out_hbm.at[pl.ds(base, bpw)])

    return k(table, idx)


def kernel(angles, sequence, W_e0, b_e0, W_e1, b_e1, W_e2, b_e2,
           W_d0, b_d0, W_d1, b_d1, W_d2, b_d2, codebook):
    nb = B // BB
    wuc = W_d2[:, :UC]
    waa = W_d2[:, UC:]
    buc = b_d2[:UC].reshape(1, UC)
    baa = b_d2[UC:].reshape(1, NAA)
    seqp = (sequence.astype(bf16).reshape(B, A, 20)
            .transpose(0, 2, 1).reshape(B, NAA))
    b0 = b_e0.reshape(1, HID)
    b1 = b_e1.reshape(1, HID)
    b2 = b_e2.reshape(1, EMB)
    bd0 = b_d0.reshape(1, HID)
    bd1 = b_d1.reshape(1, HID)
    ee = jnp.asarray(_EE, bf16)
    eo = jnp.asarray(_EO, bf16)
    eet = jnp.asarray(_EE.T, bf16)
    eot = jnp.asarray(_EO.T, bf16)
    g20 = jnp.asarray(_G20, bf16)
    g20t = jnp.asarray(_G20.T, bf16)

    C, S, enc, bi, num, wsum = pl.pallas_call(
        _encvq_body,
        grid=(nb,),
        in_specs=[
            pl.BlockSpec((BB, 384), lambda i: (i, 0)),
            pl.BlockSpec((UC, HID), lambda i: (0, 0)),
            _whole((384, UC)), _whole((384, UC)), _whole((1, HID)),
            _whole((HID, HID)), _whole((1, HID)),
            _whole((HID, EMB)), _whole((1, EMB)),
            _whole((K, EMB)),
        ],
        out_specs=[
            pl.BlockSpec((BB, 384), lambda i: (i, 0)),
            pl.BlockSpec((BB, 384), lambda i: (i, 0)),
            pl.BlockSpec((BB, EMB), lambda i: (i, 0)),
            pl.BlockSpec((1, BB), lambda i: (0, i)),
            pl.BlockSpec((1, 1), lambda i: (0, 0)),
            pl.BlockSpec((1, 1), lambda i: (0, 0)),
        ],
        out_shape=[
            jax.ShapeDtypeStruct((B, 384), f32),
            jax.ShapeDtypeStruct((B, 384), f32),
            jax.ShapeDtypeStruct((B, EMB), f32),
            jax.ShapeDtypeStruct((1, B), jnp.int32),
            jax.ShapeDtypeStruct((1, 1), f32),
            jax.ShapeDtypeStruct((1, 1), f32),
        ],
        scratch_shapes=[
            pltpu.VMEM((K, EMB), bf16),
            pltpu.VMEM((K, 1), f32),
            pltpu.VMEM((B, 384), bf16),
            pltpu.VMEM((B, 384), bf16),
            pltpu.VMEM((B, EMB), bf16),
        ],
    )(angles, W_e0, eet, eot, b0, W_e1, b1, W_e2, b2, codebook)

    quant = _sc_gather(codebook, bi[0, :])

    rec, ucr, com, aas = pl.pallas_call(
        _dec_body,
        grid=(nb,),
        in_specs=[
            pl.BlockSpec((BB, EMB), lambda i: (i, 0)),
            pl.BlockSpec((BB, EMB), lambda i: (i, 0)),
            pl.BlockSpec((BB, 384), lambda i: (i, 0)),
            pl.BlockSpec((BB, 384), lambda i: (i, 0)),
            pl.BlockSpec((BB, NAA), lambda i: (i, 0)),
            _whole((EMB, HID)), _whole((1, HID)),
            _whole((HID, HID)), _whole((1, HID)),
            _whole((HID, UC)), _whole((1, UC)),
            _whole((HID, NAA)), _whole((1, NAA)),
            _whole((UC, 384)), _whole((UC, 384)),
            _whole((NAA, A)), _whole((A, NAA)),
        ],
        out_specs=[pl.BlockSpec((1, 1), lambda i: (0, 0))] * 4,
        out_shape=[jax.ShapeDtypeStruct((1, 1), f32)] * 4,
    )(quant, enc, C, S, seqp, W_d0, bd0, W_d1, bd1, wuc, buc,
      waa, baa, ee, eo, g20, g20t)

    recon = rec[0, 0] / (B * IN_AUG)
    commit = 0.25 * com[0, 0] / (B * EMB)
    aa_loss = -aas[0, 0] / (B * A)
    uc_reg = ucr[0, 0] / (B * 384)
    dih = num[0, 0] / jnp.maximum(wsum[0, 0], 1.0)
    return recon + commit + aa_loss + 0.01 * uc_reg + 0.1 * dih


# speedup vs baseline: 1.7709x; 1.0798x over previous
import functools

import numpy as np
import jax
import jax.numpy as jnp
from jax import lax
from jax.experimental import pallas as pl
from jax.experimental.pallas import tpu as pltpu
from jax.experimental.pallas import tpu_sc as plsc

A = 128
B = 1024
HID = 1024
EMB = 512
K = 8192
UC = 768
NAA = 2560
IN_AUG = UC + NAA

BB = 256
KB = 1024

f32 = jnp.float32
bf16 = jnp.bfloat16

_EE = np.zeros((UC, 384), np.float32)
_EO = np.zeros((UC, 384), np.float32)
_EE[np.arange(384) * 2, np.arange(384)] = 1.0
_EO[np.arange(384) * 2 + 1, np.arange(384)] = 1.0
_G20 = np.zeros((NAA, A), np.float32)
_G20[np.arange(NAA), np.arange(NAA) // 20] = 1.0


def _dgT(x, y):
    return lax.dot_general(x, y, (((1,), (1,)), ((), ())),
                           preferred_element_type=f32)


def _encvq_body(ang_ref, w0_ref, eet_ref, eot_ref, b0_ref, w1_ref, b1_ref,
                w2_ref, b2_ref, cb_ref, c_ref, s_ref, e_ref, bi_ref,
                num_ref, wm_ref, cbbf_s, cn_s, cs_s, ss_s, es_s):
    i = pl.program_id(0)

    @pl.when(i == 0)
    def _():
        def prep(k, _):
            ck = cb_ref[pl.ds(k * 1024, 1024), :]
            cn_s[pl.ds(k * 1024, 1024), :] = jnp.sum(ck * ck, axis=1,
                                                     keepdims=True)
            cbbf_s[pl.ds(k * 1024, 1024), :] = ck.astype(bf16)
            return 0
        lax.fori_loop(0, K // 1024, prep, 0)

    a = ang_ref[...]
    c = jnp.cos(a)
    s = jnp.sin(a)
    c_ref[...] = c
    s_ref[...] = s
    cb_ = c.astype(bf16)
    sb_ = s.astype(bf16)
    csil = (jnp.dot(cb_, eet_ref[...], preferred_element_type=f32)
            + jnp.dot(sb_, eot_ref[...], preferred_element_type=f32))
    h = jnp.dot(csil.astype(bf16), w0_ref[...].astype(bf16),
                preferred_element_type=f32)
    h = jnp.maximum(h + b0_ref[...], 0.0)
    h = jnp.maximum(
        jnp.dot(h.astype(bf16), w1_ref[...].astype(bf16),
                preferred_element_type=f32) + b1_ref[...], 0.0)
    e = jnp.maximum(
        jnp.dot(h.astype(bf16), w2_ref[...].astype(bf16),
                preferred_element_type=f32) + b2_ref[...], 0.0)
    e_ref[...] = e
    eb = e.astype(bf16)

    ii = lax.broadcasted_iota(jnp.int32, (KB, e.shape[0]), 0)

    def chunk(k, carry):
        lm, la = carry
        ck = cbbf_s[pl.ds(k * KB, KB), :]
        cn = cn_s[pl.ds(k * KB, KB), :]
        sgn = cn - 2.0 * _dgT(ck, eb)
        cm = jnp.min(sgn, axis=0)
        cidx = jnp.min(jnp.where(sgn == cm[None, :], ii + k * KB, K), axis=0)
        upd = cm < lm
        return jnp.where(upd, cm, lm), jnp.where(upd, cidx, la)

    init = (jnp.full((e.shape[0],), jnp.inf, f32),
            jnp.zeros((e.shape[0],), jnp.int32))
    _, la = lax.fori_loop(0, K // KB, chunk, init)
    bi_ref[...] = la[None, :]

    cs_s[pl.ds(i * BB, BB), :] = cb_
    ss_s[pl.ds(i * BB, BB), :] = sb_
    es_s[pl.ds(i * BB, BB), :] = eb
    ebf = eb.astype(f32)
    e2i = jnp.sum(ebf * ebf, axis=1)

    def pair(j, carry):
        an, aw = carry
        Cj = cs_s[pl.ds(j * BB, BB), :]
        Sj = ss_s[pl.ds(j * BB, BB), :]
        Ej = es_s[pl.ds(j * BB, BB), :]
        fac = jnp.where(j == i, 1.0, 2.0)
        mc = (_dgT(cb_, Cj) + _dgT(sb_, Sj)) * (1.0 / 384.0)
        Dd = 0.5 * (1.0 - mc)
        Y = jnp.where(Dd < 0.1, 1.0, jnp.where(Dd > 0.47, 0.0, 0.5))
        wm = jnp.where(Y == 0.5, 0.0, 1.0)
        Ejf = Ej.astype(f32)
        e2j = jnp.sum(Ejf * Ejf, axis=1)
        d2 = jnp.maximum(
            e2i[:, None] + e2j[None, :] - 2.0 * _dgT(eb, Ej), 0.0)
        dn = jnp.sqrt(d2 + 1e-8)
        rl = jnp.maximum(1.0 - dn, 0.0)
        t = wm * (Y * (d2 + 1e-8) + (1.0 - Y) * rl * rl)
        return an + fac * jnp.sum(t), aw + fac * jnp.sum(wm)

    an, aw = lax.fori_loop(0, i + 1, pair,
                           (jnp.float32(0.0), jnp.float32(0.0)))

    @pl.when(i == 0)
    def _():
        num_ref[...] = an.reshape(1, 1)
        wm_ref[...] = aw.reshape(1, 1)

    @pl.when(i != 0)
    def _():
        num_ref[...] = num_ref[...] + an.reshape(1, 1)
        wm_ref[...] = wm_ref[...] + aw.reshape(1, 1)


def _dec_body(q_ref, e_ref, c_ref, s_ref, ci_ref,
              w0_ref, b0_ref, w1_ref, b1_ref, w2_ref, b2_ref,
              ee_ref, eo_ref, g20_ref, g20t_ref,
              rec_ref, ucr_ref, com_ref, aa_ref):
    first = pl.program_id(0) == 0
    q = q_ref[...]
    h = jnp.maximum(
        jnp.dot(q.astype(bf16), w0_ref[...].astype(bf16),
                preferred_element_type=f32) + b0_ref[...], 0.0)
    h = jnp.maximum(
        jnp.dot(h.astype(bf16), w1_ref[...].astype(bf16),
                preferred_element_type=f32) + b1_ref[...], 0.0)
    hb = h.astype(bf16)
    dec = (jnp.dot(hb, w2_ref[...].astype(bf16), preferred_element_type=f32)
           + b2_ref[...])
    duc = dec[:, :UC]
    aa = dec[:, UC:]
    ducb = duc.astype(bf16)
    xr = jnp.dot(ducb, ee_ref[...], preferred_element_type=f32)
    yr = jnp.dot(ducb, eo_ref[...], preferred_element_type=f32)
    C = c_ref[...]
    S = s_ref[...]
    rec = (jnp.sum((xr - C) ** 2) + jnp.sum((yr - S) ** 2)
           + jnp.sum(aa * aa))
    r2 = xr * xr + yr * yr
    ucr = jnp.sum((r2 - 1.0) ** 2)
    com = jnp.sum((e_ref[...] - q) ** 2)
    rm = jnp.max(aa, axis=1, keepdims=True)
    ex = jnp.exp(aa - rm)
    se = jnp.dot(ex.astype(bf16), g20_ref[...], preferred_element_type=f32)
    lse_sum = jnp.sum(jnp.log(se)) + jnp.sum(rm) * A
    seq = ci_ref[...]
    best = seq[:, 0:A]
    ci = jnp.zeros((seq.shape[0], A), bf16)
    for c in range(1, 20):
        sc = seq[:, c * A:(c + 1) * A]
        upd = sc > best
        best = jnp.where(upd, sc, best)
        ci = jnp.where(upd, jnp.bfloat16(c), ci)
    ciexp = jnp.dot(ci, g20t_ref[...],
                    preferred_element_type=f32)
    colpat = (lax.broadcasted_iota(jnp.int32, aa.shape, 1) % 20).astype(f32)
    sel_sum = jnp.sum(jnp.where(colpat == ciexp, aa, 0.0))
    aas = sel_sum - lse_sum

    rec = rec.reshape(1, 1)
    ucr = ucr.reshape(1, 1)
    com = com.reshape(1, 1)
    aas = aas.reshape(1, 1)

    @pl.when(first)
    def _():
        rec_ref[...] = rec
        ucr_ref[...] = ucr
        com_ref[...] = com
        aa_ref[...] = aas

    @pl.when(~first)
    def _():
        rec_ref[...] = rec_ref[...] + rec
        ucr_ref[...] = ucr_ref[...] + ucr
        com_ref[...] = com_ref[...] + com
        aa_ref[...] = aa_ref[...] + aas


def _whole(shape):
    return pl.BlockSpec(shape, lambda *_: tuple(0 for _ in shape))


def _sc_gather(table, idx):
    info = plsc.get_sparse_core_info()
    nw = info.num_cores * info.num_subcores
    bpw = B // nw
    mesh = plsc.VectorSubcoreMesh(core_axis_name="c", subcore_axis_name="s")

    @functools.partial(
        pl.kernel, mesh=mesh,
        out_type=jax.ShapeDtypeStruct((B, EMB), f32),
        scratch_types=[
            pltpu.VMEM((bpw,), jnp.int32),
            pltpu.VMEM((bpw, EMB), f32),
            pltpu.SemaphoreType.DMA,
        ],
    )
    def k(table_hbm, idx_hbm, out_hbm, idx_v, rows_v, sem):
        wid = lax.axis_index("s") * info.num_cores + lax.axis_index("c")
        base = wid * bpw
        pltpu.sync_copy(idx_hbm.at[pl.ds(base, bpw)], idx_v)
        pltpu.async_copy(table_hbm.at[idx_v], rows_v, sem).wait()
        pltpu.sync_copy(rows_v, out_hbm.at[pl.ds(base, bpw)])

    return k(table, idx)


def kernel(angles, sequence, W_e0, b_e0, W_e1, b_e1, W_e2, b_e2,
           W_d0, b_d0, W_d1, b_d1, W_d2, b_d2, codebook):
    nb = B // BB
    bd2 = b_d2.reshape(1, IN_AUG)
    seqp = (sequence.astype(bf16).reshape(B, A, 20)
            .transpose(0, 2, 1).reshape(B, NAA))
    b0 = b_e0.reshape(1, HID)
    b1 = b_e1.reshape(1, HID)
    b2 = b_e2.reshape(1, EMB)
    bd0 = b_d0.reshape(1, HID)
    bd1 = b_d1.reshape(1, HID)
    ee = jnp.asarray(_EE, bf16)
    eo = jnp.asarray(_EO, bf16)
    eet = jnp.asarray(_EE.T, bf16)
    eot = jnp.asarray(_EO.T, bf16)
    g20 = jnp.asarray(_G20, bf16)
    g20t = jnp.asarray(_G20.T, bf16)

    C, S, enc, bi, num, wsum = pl.pallas_call(
        _encvq_body,
        grid=(nb,),
        in_specs=[
            pl.BlockSpec((BB, 384), lambda i: (i, 0)),
            pl.BlockSpec((UC, HID), lambda i: (0, 0)),
            _whole((384, UC)), _whole((384, UC)), _whole((1, HID)),
            _whole((HID, HID)), _whole((1, HID)),
            _whole((HID, EMB)), _whole((1, EMB)),
            _whole((K, EMB)),
        ],
        out_specs=[
            pl.BlockSpec((BB, 384), lambda i: (i, 0)),
            pl.BlockSpec((BB, 384), lambda i: (i, 0)),
            pl.BlockSpec((BB, EMB), lambda i: (i, 0)),
            pl.BlockSpec((1, BB), lambda i: (0, i)),
            pl.BlockSpec((1, 1), lambda i: (0, 0)),
            pl.BlockSpec((1, 1), lambda i: (0, 0)),
        ],
        out_shape=[
            jax.ShapeDtypeStruct((B, 384), f32),
            jax.ShapeDtypeStruct((B, 384), f32),
            jax.ShapeDtypeStruct((B, EMB), f32),
            jax.ShapeDtypeStruct((1, B), jnp.int32),
            jax.ShapeDtypeStruct((1, 1), f32),
            jax.ShapeDtypeStruct((1, 1), f32),
        ],
        scratch_shapes=[
            pltpu.VMEM((K, EMB), bf16),
            pltpu.VMEM((K, 1), f32),
            pltpu.VMEM((B, 384), bf16),
            pltpu.VMEM((B, 384), bf16),
            pltpu.VMEM((B, EMB), bf16),
        ],
    )(angles, W_e0, eet, eot, b0, W_e1, b1, W_e2, b2, codebook)

    quant = _sc_gather(codebook, bi[0, :])

    rec, ucr, com, aas = pl.pallas_call(
        _dec_body,
        grid=(nb,),
        in_specs=[
            pl.BlockSpec((BB, EMB), lambda i: (i, 0)),
            pl.BlockSpec((BB, EMB), lambda i: (i, 0)),
            pl.BlockSpec((BB, 384), lambda i: (i, 0)),
            pl.BlockSpec((BB, 384), lambda i: (i, 0)),
            pl.BlockSpec((BB, NAA), lambda i: (i, 0)),
            _whole((EMB, HID)), _whole((1, HID)),
            _whole((HID, HID)), _whole((1, HID)),
            _whole((HID, IN_AUG)), _whole((1, IN_AUG)),
            _whole((UC, 384)), _whole((UC, 384)),
            _whole((NAA, A)), _whole((A, NAA)),
        ],
        out_specs=[pl.BlockSpec((1, 1), lambda i: (0, 0))] * 4,
        out_shape=[jax.ShapeDtypeStruct((1, 1), f32)] * 4,
    )(quant, enc, C, S, seqp, W_d0, bd0, W_d1, bd1, W_d2, bd2,
      ee, eo, g20, g20t)

    recon = rec[0, 0] / (B * IN_AUG)
    commit = 0.25 * com[0, 0] / (B * EMB)
    aa_loss = -aas[0, 0] / (B * A)
    uc_reg = ucr[0, 0] / (B * 384)
    dih = num[0, 0] / jnp.maximum(wsum[0, 0], 1.0)
    return recon + commit + aa_loss + 0.01 * uc_reg + 0.1 * dih


# speedup vs baseline: 1.8129x; 1.0237x over previous
import functools

import numpy as np
import jax
import jax.numpy as jnp
from jax import lax
from jax.experimental import pallas as pl
from jax.experimental.pallas import tpu as pltpu
from jax.experimental.pallas import tpu_sc as plsc

A = 128
B = 1024
HID = 1024
EMB = 512
K = 8192
UC = 768
NAA = 2560
IN_AUG = UC + NAA

BB = 256
KB = 2048

f32 = jnp.float32
bf16 = jnp.bfloat16

_EE = np.zeros((UC, 384), np.float32)
_EO = np.zeros((UC, 384), np.float32)
_EE[np.arange(384) * 2, np.arange(384)] = 1.0
_EO[np.arange(384) * 2 + 1, np.arange(384)] = 1.0
_G20 = np.zeros((NAA, A), np.float32)
_G20[np.arange(NAA), np.arange(NAA) // 20] = 1.0


def _dgT(x, y):
    return lax.dot_general(x, y, (((1,), (1,)), ((), ())),
                           preferred_element_type=f32)


def _encvq_body(ang_ref, w0_ref, eet_ref, eot_ref, b0_ref, w1_ref, b1_ref,
                w2_ref, b2_ref, cb_ref, c_ref, s_ref, e_ref, bi_ref,
                num_ref, wm_ref, cbbf_s, cn_s, cs_s, ss_s, es_s):
    i = pl.program_id(0)

    @pl.when(i == 0)
    def _():
        def prep(k, _):
            ck = cb_ref[pl.ds(k * 1024, 1024), :]
            cn_s[pl.ds(k * 1024, 1024), :] = jnp.sum(ck * ck, axis=1,
                                                     keepdims=True)
            cbbf_s[pl.ds(k * 1024, 1024), :] = ck.astype(bf16)
            return 0
        lax.fori_loop(0, K // 1024, prep, 0)

    a = ang_ref[...]
    c = jnp.cos(a)
    s = jnp.sin(a)
    c_ref[...] = c
    s_ref[...] = s
    cb_ = c.astype(bf16)
    sb_ = s.astype(bf16)
    csil = (jnp.dot(cb_, eet_ref[...], preferred_element_type=f32)
            + jnp.dot(sb_, eot_ref[...], preferred_element_type=f32))
    h = jnp.dot(csil.astype(bf16), w0_ref[...].astype(bf16),
                preferred_element_type=f32)
    h = jnp.maximum(h + b0_ref[...], 0.0)
    h = jnp.maximum(
        jnp.dot(h.astype(bf16), w1_ref[...].astype(bf16),
                preferred_element_type=f32) + b1_ref[...], 0.0)
    e = jnp.maximum(
        jnp.dot(h.astype(bf16), w2_ref[...].astype(bf16),
                preferred_element_type=f32) + b2_ref[...], 0.0)
    e_ref[...] = e
    eb = e.astype(bf16)

    ii = lax.broadcasted_iota(jnp.int32, (KB, e.shape[0]), 0)

    def chunk(k, carry):
        lm, la = carry
        ck = cbbf_s[pl.ds(k * KB, KB), :]
        cn = cn_s[pl.ds(k * KB, KB), :]
        sgn = cn - 2.0 * _dgT(ck, eb)
        cm = jnp.min(sgn, axis=0)
        cidx = jnp.min(jnp.where(sgn == cm[None, :], ii + k * KB, K), axis=0)
        upd = cm < lm
        return jnp.where(upd, cm, lm), jnp.where(upd, cidx, la)

    init = (jnp.full((e.shape[0],), jnp.inf, f32),
            jnp.zeros((e.shape[0],), jnp.int32))
    _, la = lax.fori_loop(0, K // KB, chunk, init)
    bi_ref[...] = la[None, :]

    cs_s[pl.ds(i * BB, BB), :] = cb_
    ss_s[pl.ds(i * BB, BB), :] = sb_
    es_s[pl.ds(i * BB, BB), :] = eb
    ebf = eb.astype(f32)
    e2i = jnp.sum(ebf * ebf, axis=1)

    def pair(j, carry):
        an, aw = carry
        Cj = cs_s[pl.ds(j * BB, BB), :]
        Sj = ss_s[pl.ds(j * BB, BB), :]
        Ej = es_s[pl.ds(j * BB, BB), :]
        fac = jnp.where(j == i, 1.0, 2.0)
        mc = (_dgT(cb_, Cj) + _dgT(sb_, Sj)) * (1.0 / 384.0)
        Dd = 0.5 * (1.0 - mc)
        Y = jnp.where(Dd < 0.1, 1.0, jnp.where(Dd > 0.47, 0.0, 0.5))
        wm = jnp.where(Y == 0.5, 0.0, 1.0)
        Ejf = Ej.astype(f32)
        e2j = jnp.sum(Ejf * Ejf, axis=1)
        d2 = jnp.maximum(
            e2i[:, None] + e2j[None, :] - 2.0 * _dgT(eb, Ej), 0.0)
        dn = jnp.sqrt(d2 + 1e-8)
        rl = jnp.maximum(1.0 - dn, 0.0)
        t = wm * (Y * (d2 + 1e-8) + (1.0 - Y) * rl * rl)
        return an + fac * jnp.sum(t), aw + fac * jnp.sum(wm)

    an, aw = lax.fori_loop(0, i + 1, pair,
                           (jnp.float32(0.0), jnp.float32(0.0)))

    @pl.when(i == 0)
    def _():
        num_ref[...] = an.reshape(1, 1)
        wm_ref[...] = aw.reshape(1, 1)

    @pl.when(i != 0)
    def _():
        num_ref[...] = num_ref[...] + an.reshape(1, 1)
        wm_ref[...] = wm_ref[...] + aw.reshape(1, 1)


def _dec_body(q_ref, e_ref, c_ref, s_ref, ci_ref,
              w0_ref, b0_ref, w1_ref, b1_ref, w2_ref, b2_ref,
              ee_ref, eo_ref, g20_ref, g20t_ref,
              rec_ref, ucr_ref, com_ref, aa_ref):
    first = pl.program_id(0) == 0
    q = q_ref[...]
    h = jnp.maximum(
        jnp.dot(q.astype(bf16), w0_ref[...].astype(bf16),
                preferred_element_type=f32) + b0_ref[...], 0.0)
    h = jnp.maximum(
        jnp.dot(h.astype(bf16), w1_ref[...].astype(bf16),
                preferred_element_type=f32) + b1_ref[...], 0.0)
    hb = h.astype(bf16)
    dec = (jnp.dot(hb, w2_ref[...].astype(bf16), preferred_element_type=f32)
           + b2_ref[...])
    duc = dec[:, :UC]
    aa = dec[:, UC:]
    ducb = duc.astype(bf16)
    xr = jnp.dot(ducb, ee_ref[...], preferred_element_type=f32)
    yr = jnp.dot(ducb, eo_ref[...], preferred_element_type=f32)
    C = c_ref[...]
    S = s_ref[...]
    rec = (jnp.sum((xr - C) ** 2) + jnp.sum((yr - S) ** 2)
           + jnp.sum(aa * aa))
    r2 = xr * xr + yr * yr
    ucr = jnp.sum((r2 - 1.0) ** 2)
    com = jnp.sum((e_ref[...] - q) ** 2)
    rm = jnp.max(aa, axis=1, keepdims=True)
    ex = jnp.exp(aa - rm)
    se = jnp.dot(ex.astype(bf16), g20_ref[...], preferred_element_type=f32)
    lse_sum = jnp.sum(jnp.log(se)) + jnp.sum(rm) * A
    seq = ci_ref[...]
    best = seq[:, 0:A]
    ci = jnp.zeros((seq.shape[0], A), bf16)
    for c in range(1, 20):
        sc = seq[:, c * A:(c + 1) * A]
        upd = sc > best
        best = jnp.where(upd, sc, best)
        ci = jnp.where(upd, jnp.bfloat16(c), ci)
    ciexp = jnp.dot(ci, g20t_ref[...],
                    preferred_element_type=f32)
    colpat = (lax.broadcasted_iota(jnp.int32, aa.shape, 1) % 20).astype(f32)
    sel_sum = jnp.sum(jnp.where(colpat == ciexp, aa, 0.0))
    aas = sel_sum - lse_sum

    rec = rec.reshape(1, 1)
    ucr = ucr.reshape(1, 1)
    com = com.reshape(1, 1)
    aas = aas.reshape(1, 1)

    @pl.when(first)
    def _():
        rec_ref[...] = rec
        ucr_ref[...] = ucr
        com_ref[...] = com
        aa_ref[...] = aas

    @pl.when(~first)
    def _():
        rec_ref[...] = rec_ref[...] + rec
        ucr_ref[...] = ucr_ref[...] + ucr
        com_ref[...] = com_ref[...] + com
        aa_ref[...] = aa_ref[...] + aas


def _whole(shape):
    return pl.BlockSpec(shape, lambda *_: tuple(0 for _ in shape))


def _sc_gather(table, idx):
    info = plsc.get_sparse_core_info()
    nw = info.num_cores * info.num_subcores
    bpw = B // nw
    mesh = plsc.VectorSubcoreMesh(core_axis_name="c", subcore_axis_name="s")

    @functools.partial(
        pl.kernel, mesh=mesh,
        out_type=jax.ShapeDtypeStruct((B, EMB), f32),
        scratch_types=[
            pltpu.VMEM((bpw,), jnp.int32),
            pltpu.VMEM((bpw, EMB), f32),
            pltpu.SemaphoreType.DMA,
        ],
    )
    def k(table_hbm, idx_hbm, out_hbm, idx_v, rows_v, sem):
        wid = lax.axis_index("s") * info.num_cores + lax.axis_index("c")
        base = wid * bpw
        pltpu.sync_copy(idx_hbm.at[pl.ds(base, bpw)], idx_v)
        pltpu.async_copy(table_hbm.at[idx_v], rows_v, sem).wait()
        pltpu.sync_copy(rows_v, out_hbm.at[pl.ds(base, bpw)])

    return k(table, idx)


def kernel(angles, sequence, W_e0, b_e0, W_e1, b_e1, W_e2, b_e2,
           W_d0, b_d0, W_d1, b_d1, W_d2, b_d2, codebook):
    nb = B // BB
    bd2 = b_d2.reshape(1, IN_AUG)
    seqp = (sequence.astype(bf16).reshape(B, A, 20)
            .transpose(0, 2, 1).reshape(B, NAA))
    b0 = b_e0.reshape(1, HID)
    b1 = b_e1.reshape(1, HID)
    b2 = b_e2.reshape(1, EMB)
    bd0 = b_d0.reshape(1, HID)
    bd1 = b_d1.reshape(1, HID)
    ee = jnp.asarray(_EE, bf16)
    eo = jnp.asarray(_EO, bf16)
    eet = jnp.asarray(_EE.T, bf16)
    eot = jnp.asarray(_EO.T, bf16)
    g20 = jnp.asarray(_G20, bf16)
    g20t = jnp.asarray(_G20.T, bf16)

    C, S, enc, bi, num, wsum = pl.pallas_call(
        _encvq_body,
        grid=(nb,),
        in_specs=[
            pl.BlockSpec((BB, 384), lambda i: (i, 0)),
            pl.BlockSpec((UC, HID), lambda i: (0, 0)),
            _whole((384, UC)), _whole((384, UC)), _whole((1, HID)),
            _whole((HID, HID)), _whole((1, HID)),
            _whole((HID, EMB)), _whole((1, EMB)),
            _whole((K, EMB)),
        ],
        out_specs=[
            pl.BlockSpec((BB, 384), lambda i: (i, 0)),
            pl.BlockSpec((BB, 384), lambda i: (i, 0)),
            pl.BlockSpec((BB, EMB), lambda i: (i, 0)),
            pl.BlockSpec((1, BB), lambda i: (0, i)),
            pl.BlockSpec((1, 1), lambda i: (0, 0)),
            pl.BlockSpec((1, 1), lambda i: (0, 0)),
        ],
        out_shape=[
            jax.ShapeDtypeStruct((B, 384), f32),
            jax.ShapeDtypeStruct((B, 384), f32),
            jax.ShapeDtypeStruct((B, EMB), f32),
            jax.ShapeDtypeStruct((1, B), jnp.int32),
            jax.ShapeDtypeStruct((1, 1), f32),
            jax.ShapeDtypeStruct((1, 1), f32),
        ],
        scratch_shapes=[
            pltpu.VMEM((K, EMB), bf16),
            pltpu.VMEM((K, 1), f32),
            pltpu.VMEM((B, 384), bf16),
            pltpu.VMEM((B, 384), bf16),
            pltpu.VMEM((B, EMB), bf16),
        ],
    )(angles, W_e0, eet, eot, b0, W_e1, b1, W_e2, b2, codebook)

    quant = _sc_gather(codebook, bi[0, :])

    rec, ucr, com, aas = pl.pallas_call(
        _dec_body,
        grid=(nb,),
        in_specs=[
            pl.BlockSpec((BB, EMB), lambda i: (i, 0)),
            pl.BlockSpec((BB, EMB), lambda i: (i, 0)),
            pl.BlockSpec((BB, 384), lambda i: (i, 0)),
            pl.BlockSpec((BB, 384), lambda i: (i, 0)),
            pl.BlockSpec((BB, NAA), lambda i: (i, 0)),
            _whole((EMB, HID)), _whole((1, HID)),
            _whole((HID, HID)), _whole((1, HID)),
            _whole((HID, IN_AUG)), _whole((1, IN_AUG)),
            _whole((UC, 384)), _whole((UC, 384)),
            _whole((NAA, A)), _whole((A, NAA)),
        ],
        out_specs=[pl.BlockSpec((1, 1), lambda i: (0, 0))] * 4,
        out_shape=[jax.ShapeDtypeStruct((1, 1), f32)] * 4,
    )(quant, enc, C, S, seqp, W_d0, bd0, W_d1, bd1, W_d2, bd2,
      ee, eo, g20, g20t)

    recon = rec[0, 0] / (B * IN_AUG)
    commit = 0.25 * com[0, 0] / (B * EMB)
    aa_loss = -aas[0, 0] / (B * A)
    uc_reg = ucr[0, 0] / (B * 384)
    dih = num[0, 0] / jnp.maximum(wsum[0, 0], 1.0)
    return recon + commit + aa_loss + 0.01 * uc_reg + 0.1 * dih


# speedup vs baseline: 1.8518x; 1.0215x over previous
import functools

import numpy as np
import jax
import jax.numpy as jnp
from jax import lax
from jax.experimental import pallas as pl
from jax.experimental.pallas import tpu as pltpu
from jax.experimental.pallas import tpu_sc as plsc

A = 128
B = 1024
HID = 1024
EMB = 512
K = 8192
UC = 768
NAA = 2560
IN_AUG = UC + NAA

BB = 256
KB = 2048

f32 = jnp.float32
bf16 = jnp.bfloat16

_EE = np.zeros((UC, 384), np.float32)
_EO = np.zeros((UC, 384), np.float32)
_EE[np.arange(384) * 2, np.arange(384)] = 1.0
_EO[np.arange(384) * 2 + 1, np.arange(384)] = 1.0
_G20 = np.zeros((NAA, A), np.float32)
_G20[np.arange(NAA), np.arange(NAA) // 20] = 1.0


def _dgT(x, y):
    return lax.dot_general(x, y, (((1,), (1,)), ((), ())),
                           preferred_element_type=f32)


def _encvq_body(ang_ref, w0_ref, eet_ref, eot_ref, b0_ref, w1_ref, b1_ref,
                w2_ref, b2_ref, cb_ref, c_ref, s_ref, e_ref, bi_ref,
                num_ref, wm_ref, cbbf_s, cn_s, cs_s, ss_s, es_s):
    i = pl.program_id(0)

    @pl.when(i == 0)
    def _():
        def prep(k, _):
            ck = cb_ref[pl.ds(k * 1024, 1024), :]
            cn_s[pl.ds(k * 1024, 1024), :] = jnp.sum(ck * ck, axis=1,
                                                     keepdims=True)
            cbbf_s[pl.ds(k * 1024, 1024), :] = ck.astype(bf16)
            return 0
        lax.fori_loop(0, K // 1024, prep, 0)

    a = ang_ref[...]
    c = jnp.cos(a)
    s = jnp.sin(a)
    c_ref[...] = c
    s_ref[...] = s
    cb_ = c.astype(bf16)
    sb_ = s.astype(bf16)
    csil = (jnp.dot(cb_, eet_ref[...], preferred_element_type=f32)
            + jnp.dot(sb_, eot_ref[...], preferred_element_type=f32))
    h = jnp.dot(csil.astype(bf16), w0_ref[...].astype(bf16),
                preferred_element_type=f32)
    h = jnp.maximum(h + b0_ref[...], 0.0)
    h = jnp.maximum(
        jnp.dot(h.astype(bf16), w1_ref[...].astype(bf16),
                preferred_element_type=f32) + b1_ref[...], 0.0)
    e = jnp.maximum(
        jnp.dot(h.astype(bf16), w2_ref[...].astype(bf16),
                preferred_element_type=f32) + b2_ref[...], 0.0)
    e_ref[...] = e
    eb = e.astype(bf16)

    ii = lax.broadcasted_iota(jnp.int32, (KB, e.shape[0]), 0)

    def chunk(k, carry):
        lm, la = carry
        ck = cbbf_s[pl.ds(k * KB, KB), :]
        cn = cn_s[pl.ds(k * KB, KB), :]
        sgn = cn - 2.0 * _dgT(ck, eb)
        cm = jnp.min(sgn, axis=0)
        cidx = jnp.min(jnp.where(sgn == cm[None, :], ii + k * KB, K), axis=0)
        upd = cm < lm
        return jnp.where(upd, cm, lm), jnp.where(upd, cidx, la)

    init = (jnp.full((e.shape[0],), jnp.inf, f32),
            jnp.zeros((e.shape[0],), jnp.int32))
    _, la = lax.fori_loop(0, K // KB, chunk, init)
    bi_ref[...] = la[None, :]

    cs_s[pl.ds(i * BB, BB), :] = cb_
    ss_s[pl.ds(i * BB, BB), :] = sb_
    es_s[pl.ds(i * BB, BB), :] = eb
    ebf = eb.astype(f32)
    e2i = jnp.sum(ebf * ebf, axis=1)

    def pair(j, carry):
        an, aw = carry
        Cj = cs_s[pl.ds(j * BB, BB), :]
        Sj = ss_s[pl.ds(j * BB, BB), :]
        Ej = es_s[pl.ds(j * BB, BB), :]
        fac = jnp.where(j == i, 1.0, 2.0)
        mc = (_dgT(cb_, Cj) + _dgT(sb_, Sj)) * (1.0 / 384.0)
        Dd = 0.5 * (1.0 - mc)
        Y = jnp.where(Dd < 0.1, 1.0, jnp.where(Dd > 0.47, 0.0, 0.5))
        wm = jnp.where(Y == 0.5, 0.0, 1.0)
        Ejf = Ej.astype(f32)
        e2j = jnp.sum(Ejf * Ejf, axis=1)
        d2 = jnp.maximum(
            e2i[:, None] + e2j[None, :] - 2.0 * _dgT(eb, Ej), 0.0)
        dn = jnp.sqrt(d2 + 1e-8)
        rl = jnp.maximum(1.0 - dn, 0.0)
        t = wm * (Y * (d2 + 1e-8) + (1.0 - Y) * rl * rl)
        return an + fac * jnp.sum(t), aw + fac * jnp.sum(wm)

    an, aw = lax.fori_loop(0, i + 1, pair,
                           (jnp.float32(0.0), jnp.float32(0.0)))

    @pl.when(i == 0)
    def _():
        num_ref[...] = an.reshape(1, 1)
        wm_ref[...] = aw.reshape(1, 1)

    @pl.when(i != 0)
    def _():
        num_ref[...] = num_ref[...] + an.reshape(1, 1)
        wm_ref[...] = wm_ref[...] + aw.reshape(1, 1)


def _dec_body(q_ref, e_ref, c_ref, s_ref, ci_ref,
              w0_ref, b0_ref, w1_ref, b1_ref, w2_ref, b2_ref,
              ee_ref, eo_ref, g20_ref, g20t_ref,
              rec_ref, ucr_ref, com_ref, aa_ref):
    first = pl.program_id(0) == 0
    q = q_ref[...]
    h = jnp.maximum(
        jnp.dot(q.astype(bf16), w0_ref[...].astype(bf16),
                preferred_element_type=f32) + b0_ref[...], 0.0)
    h = jnp.maximum(
        jnp.dot(h.astype(bf16), w1_ref[...].astype(bf16),
                preferred_element_type=f32) + b1_ref[...], 0.0)
    hb = h.astype(bf16)
    dec = (jnp.dot(hb, w2_ref[...].astype(bf16), preferred_element_type=f32)
           + b2_ref[...])
    duc = dec[:, :UC]
    aa = dec[:, UC:]
    ducb = duc.astype(bf16)
    xr = jnp.dot(ducb, ee_ref[...], preferred_element_type=f32)
    yr = jnp.dot(ducb, eo_ref[...], preferred_element_type=f32)
    C = c_ref[...]
    S = s_ref[...]
    rec = (jnp.sum((xr - C) ** 2) + jnp.sum((yr - S) ** 2)
           + jnp.sum(aa * aa))
    r2 = xr * xr + yr * yr
    ucr = jnp.sum((r2 - 1.0) ** 2)
    com = jnp.sum((e_ref[...] - q) ** 2)
    rm = jnp.max(aa, axis=1, keepdims=True)
    ex = jnp.exp(aa - rm)
    se = jnp.dot(ex.astype(bf16), g20_ref[...], preferred_element_type=f32)
    lse_sum = jnp.sum(jnp.log(se)) + jnp.sum(rm) * A
    seq = ci_ref[...]
    best = seq[:, 0:A]
    ci = jnp.zeros((seq.shape[0], A), bf16)
    for c in range(1, 20):
        sc = seq[:, c * A:(c + 1) * A]
        upd = sc > best
        best = jnp.where(upd, sc, best)
        ci = jnp.where(upd, jnp.bfloat16(c), ci)
    ciexp = jnp.dot(ci, g20t_ref[...],
                    preferred_element_type=f32)
    colpat = (lax.broadcasted_iota(jnp.int32, aa.shape, 1) % 20).astype(f32)
    sel_sum = jnp.sum(jnp.where(colpat == ciexp, aa, 0.0))
    aas = sel_sum - lse_sum

    rec = rec.reshape(1, 1)
    ucr = ucr.reshape(1, 1)
    com = com.reshape(1, 1)
    aas = aas.reshape(1, 1)

    @pl.when(first)
    def _():
        rec_ref[...] = rec
        ucr_ref[...] = ucr
        com_ref[...] = com
        aa_ref[...] = aas

    @pl.when(~first)
    def _():
        rec_ref[...] = rec_ref[...] + rec
        ucr_ref[...] = ucr_ref[...] + ucr
        com_ref[...] = com_ref[...] + com
        aa_ref[...] = aa_ref[...] + aas


def _whole(shape):
    return pl.BlockSpec(shape, lambda *_: tuple(0 for _ in shape))


def _sc_gather(table, idx):
    info = plsc.get_sparse_core_info()
    nw = info.num_cores * info.num_subcores
    bpw = B // nw
    mesh = plsc.VectorSubcoreMesh(core_axis_name="c", subcore_axis_name="s")

    @functools.partial(
        pl.kernel, mesh=mesh,
        out_type=jax.ShapeDtypeStruct((B, EMB), f32),
        scratch_types=[
            pltpu.VMEM((bpw,), jnp.int32),
            pltpu.VMEM((bpw, EMB), f32),
            pltpu.SemaphoreType.DMA,
        ],
    )
    def k(table_hbm, idx_hbm, out_hbm, idx_v, rows_v, sem):
        wid = lax.axis_index("s") * info.num_cores + lax.axis_index("c")
        base = wid * bpw
        pltpu.sync_copy(idx_hbm.at[pl.ds(base, bpw)], idx_v)
        pltpu.async_copy(table_hbm.at[idx_v], rows_v, sem).wait()
        pltpu.sync_copy(rows_v, out_hbm.at[pl.ds(base, bpw)])

    return k(table, idx)


def kernel(angles, sequence, W_e0, b_e0, W_e1, b_e1, W_e2, b_e2,
           W_d0, b_d0, W_d1, b_d1, W_d2, b_d2, codebook):
    nb = B // BB
    bd2 = b_d2.reshape(1, IN_AUG)
    aperm = jnp.asarray((np.arange(NAA) % A) * 20 + np.arange(NAA) // A)
    seqp = jnp.take(sequence, aperm, axis=1).astype(bf16)
    b0 = b_e0.reshape(1, HID)
    b1 = b_e1.reshape(1, HID)
    b2 = b_e2.reshape(1, EMB)
    bd0 = b_d0.reshape(1, HID)
    bd1 = b_d1.reshape(1, HID)
    ee = jnp.asarray(_EE, bf16)
    eo = jnp.asarray(_EO, bf16)
    eet = jnp.asarray(_EE.T, bf16)
    eot = jnp.asarray(_EO.T, bf16)
    g20 = jnp.asarray(_G20, bf16)
    g20t = jnp.asarray(_G20.T, bf16)

    C, S, enc, bi, num, wsum = pl.pallas_call(
        _encvq_body,
        grid=(nb,),
        in_specs=[
            pl.BlockSpec((BB, 384), lambda i: (i, 0)),
            pl.BlockSpec((UC, HID), lambda i: (0, 0)),
            _whole((384, UC)), _whole((384, UC)), _whole((1, HID)),
            _whole((HID, HID)), _whole((1, HID)),
            _whole((HID, EMB)), _whole((1, EMB)),
            _whole((K, EMB)),
        ],
        out_specs=[
            pl.BlockSpec((BB, 384), lambda i: (i, 0)),
            pl.BlockSpec((BB, 384), lambda i: (i, 0)),
            pl.BlockSpec((BB, EMB), lambda i: (i, 0)),
            pl.BlockSpec((1, BB), lambda i: (0, i)),
            pl.BlockSpec((1, 1), lambda i: (0, 0)),
            pl.BlockSpec((1, 1), lambda i: (0, 0)),
        ],
        out_shape=[
            jax.ShapeDtypeStruct((B, 384), f32),
            jax.ShapeDtypeStruct((B, 384), f32),
            jax.ShapeDtypeStruct((B, EMB), f32),
            jax.ShapeDtypeStruct((1, B), jnp.int32),
            jax.ShapeDtypeStruct((1, 1), f32),
            jax.ShapeDtypeStruct((1, 1), f32),
        ],
        scratch_shapes=[
            pltpu.VMEM((K, EMB), bf16),
            pltpu.VMEM((K, 1), f32),
            pltpu.VMEM((B, 384), bf16),
            pltpu.VMEM((B, 384), bf16),
            pltpu.VMEM((B, EMB), bf16),
        ],
    )(angles, W_e0, eet, eot, b0, W_e1, b1, W_e2, b2, codebook)

    quant = _sc_gather(codebook, bi[0, :])

    rec, ucr, com, aas = pl.pallas_call(
        _dec_body,
        grid=(nb,),
        in_specs=[
            pl.BlockSpec((BB, EMB), lambda i: (i, 0)),
            pl.BlockSpec((BB, EMB), lambda i: (i, 0)),
            pl.BlockSpec((BB, 384), lambda i: (i, 0)),
            pl.BlockSpec((BB, 384), lambda i: (i, 0)),
            pl.BlockSpec((BB, NAA), lambda i: (i, 0)),
            _whole((EMB, HID)), _whole((1, HID)),
            _whole((HID, HID)), _whole((1, HID)),
            _whole((HID, IN_AUG)), _whole((1, IN_AUG)),
            _whole((UC, 384)), _whole((UC, 384)),
            _whole((NAA, A)), _whole((A, NAA)),
        ],
        out_specs=[pl.BlockSpec((1, 1), lambda i: (0, 0))] * 4,
        out_shape=[jax.ShapeDtypeStruct((1, 1), f32)] * 4,
    )(quant, enc, C, S, seqp, W_d0, bd0, W_d1, bd1, W_d2, bd2,
      ee, eo, g20, g20t)

    recon = rec[0, 0] / (B * IN_AUG)
    commit = 0.25 * com[0, 0] / (B * EMB)
    aa_loss = -aas[0, 0] / (B * A)
    uc_reg = ucr[0, 0] / (B * 384)
    dih = num[0, 0] / jnp.maximum(wsum[0, 0], 1.0)
    return recon + commit + aa_loss + 0.01 * uc_reg + 0.1 * dih
